# baseline (speedup 1.0000x reference)
import dataclasses

import numpy as _np

import jax
import jax.numpy as jnp
from jax import lax
from jax.experimental import pallas as pl
from jax.experimental.pallas import tpu as pltpu
from jax.experimental.pallas import tpu_sc as plsc

_N = 10000
_DEG = 32
_F = 128
_OUT_F = 128
_E = _N * _DEG

_NW = 32
_CH = 4
_EPC = _CH * _DEG
_NCHUNKS = _N // _CH
_NITER = 80
_SUP = 2
_NSUP = _NITER // _SUP

_LANES = 16
_FCH = _F // _LANES


def _agg_body(idx_hbm, val_hbm, x_hbm, y_hbm,
              idx0, val0, rows0, out0, semi0, semv0, semg0, semo0,
              idx1, val1, rows1, out1, semi1, semv1, semg1, semo1):
    wid = lax.axis_index("s") * 2 + lax.axis_index("c")

    def chunk_of(g):
        return jnp.minimum(jnp.minimum(g, _NITER - 1) * _NW + wid,
                           _NCHUNKS - 1)

    def start_i(s, idx_v, semi):
        for q in range(_SUP):
            pltpu.async_copy(
                idx_hbm.at[pl.ds(chunk_of(s * _SUP + q) * _EPC, _EPC)],
                idx_v.at[q], semi)

    def wait_i(idx_v, semi):
        for q in range(_SUP):
            pltpu.make_async_copy(idx_hbm.at[pl.ds(0, _EPC)],
                                  idx_v.at[q], semi).wait()

    def start_v(s, val_v, semv):
        for q in range(_SUP):
            pltpu.async_copy(
                val_hbm.at[pl.ds(chunk_of(s * _SUP + q) * _EPC, _EPC)],
                val_v.at[q], semv)

    def wait_v(val_v, semv):
        for q in range(_SUP):
            pltpu.make_async_copy(val_hbm.at[pl.ds(0, _EPC)],
                                  val_v.at[q], semv).wait()

    def start_g(idx_v, rows_v, semg):
        for q in range(_SUP):
            pltpu.async_copy(x_hbm.at[idx_v.at[q]],
                             rows_v.at[pl.ds(q * _EPC, _EPC)], semg)

    def wait_g(idx_v, rows_v, semg):
        for q in range(_SUP):
            pltpu.make_async_copy(x_hbm.at[idx_v.at[q]],
                                  rows_v.at[pl.ds(q * _EPC, _EPC)],
                                  semg).wait()

    def start_o(s, out_v, semo):
        for q in range(_SUP):
            pltpu.async_copy(
                out_v.at[pl.ds(q * _CH, _CH)],
                y_hbm.at[pl.ds(chunk_of(s * _SUP + q) * _CH, _CH)], semo)

    def wait_o(out_v, semo):
        for q in range(_SUP):
            pltpu.make_async_copy(out_v.at[pl.ds(0, _CH)],
                                  y_hbm.at[pl.ds(0, _CH)], semo).wait()

    def bcast_gather(src, idx):
        return lax.gather(
            src, idx,
            dimension_numbers=lax.GatherDimensionNumbers(
                offset_dims=(), collapsed_slice_dims=(0,),
                start_index_map=(0,)),
            slice_sizes=(1,),
            mode=lax.GatherScatterMode.PROMISE_IN_BOUNDS)

    def compute(rows_v, val_v, out_v):
        for q in range(_SUP):
            for n in range(_CH):
                def group(h, accs, q=q, n=n):
                    vv = val_v[q, pl.ds(n * _DEG + (h // 2) * _LANES,
                                        _LANES)]
                    sub = (h % 2) * 8
                    base = n * _DEG + h * 8
                    for k in range(8):
                        v = bcast_gather(
                            vv, jnp.full((_LANES, 1), sub + k, jnp.int32))
                        j = q * _EPC + base + k
                        accs = tuple(
                            accs[fc]
                            + v * rows_v[j, pl.ds(fc * _LANES, _LANES)]
                            for fc in range(_FCH))
                    return accs

                accs = lax.fori_loop(
                    0, _DEG // 8, group,
                    tuple(jnp.zeros((_LANES,), jnp.float32)
                          for _ in range(_FCH)))
                for fc in range(_FCH):
                    out_v[q * _CH + n, pl.ds(fc * _LANES, _LANES)] = accs[fc]

    sets = ((idx0, val0, rows0, out0, semi0, semv0, semg0, semo0),
            (idx1, val1, rows1, out1, semi1, semv1, semg1, semo1))

    start_i(0, idx0, semi0)
    start_i(1, idx1, semi1)
    start_v(0, val0, semv0)
    start_v(1, val1, semv1)
    wait_i(idx0, semi0)
    start_g(idx0, rows0, semg0)

    def step(s, a, b):
        idx_a, val_a, rows_a, out_a, semi_a, semv_a, semg_a, semo_a = a
        idx_b, val_b, rows_b, out_b, semi_b, semv_b, semg_b, semo_b = b
        wait_i(idx_b, semi_b)
        start_g(idx_b, rows_b, semg_b)
        wait_g(idx_a, rows_a, semg_a)
        start_i(s + 2, idx_a, semi_a)
        wait_v(val_a, semv_a)

        @pl.when(s >= 2)
        def _():
            wait_o(out_a, semo_a)

        compute(rows_a, val_a, out_a)
        start_o(s, out_a, semo_a)
        start_v(s + 2, val_a, semv_a)

    @pl.loop(0, _NSUP, step=2)
    def _(s):
        step(s, sets[0], sets[1])
        step(s + 1, sets[1], sets[0])

    wait_g(idx0, rows0, semg0)
    wait_i(idx1, semi1)
    wait_v(val0, semv0)
    wait_v(val1, semv1)
    wait_o(out0, semo0)
    wait_o(out1, semo1)


@jax.jit
def _aggregate(col_idx, values, X):
    mesh = plsc.VectorSubcoreMesh(core_axis_name="c", subcore_axis_name="s")
    cp = pltpu.CompilerParams()
    if "needs_layout_passes" in pltpu.CompilerParams.__dataclass_fields__:
        cp = dataclasses.replace(cp, needs_layout_passes=False)
    buf_set = [
        pltpu.VMEM((_SUP, _EPC), jnp.int32),
        pltpu.VMEM((_SUP, _EPC), jnp.float32),
        pltpu.VMEM((_SUP * _EPC, _F), jnp.float32),
        pltpu.VMEM((_SUP * _CH, _F), jnp.float32),
        pltpu.SemaphoreType.DMA,
        pltpu.SemaphoreType.DMA,
        pltpu.SemaphoreType.DMA,
        pltpu.SemaphoreType.DMA,
    ]
    return pl.kernel(
        _agg_body,
        out_type=jax.ShapeDtypeStruct((_N, _F), jnp.float32),
        mesh=mesh,
        scratch_types=buf_set + buf_set,
        compiler_params=cp,
    )(col_idx, values, X)


def _mm_body(y_ref, w_ref, o_ref):
    o_ref[...] = jnp.dot(y_ref[...], w_ref[...],
                         preferred_element_type=jnp.float32,
                         precision=lax.Precision.HIGHEST)


_MB = 2000


@jax.jit
def _matmul(Y, W):
    return pl.pallas_call(
        _mm_body,
        grid=(_N // _MB,),
        in_specs=[
            pl.BlockSpec((_MB, _F), lambda i: (i, 0)),
            pl.BlockSpec((_F, _OUT_F), lambda i: (0, 0)),
        ],
        out_specs=pl.BlockSpec((_MB, _OUT_F), lambda i: (i, 0)),
        out_shape=jax.ShapeDtypeStruct((_N, _OUT_F), jnp.float32),
    )(Y, W)


def kernel(row_ptr, col_idx, values, X, num_neighbors, W):
    Y = _aggregate(col_idx, values, X)
    return _matmul(Y, W)


# speedup vs baseline: 1.0579x; 1.0579x over previous
import dataclasses

import numpy as _np

import jax
import jax.numpy as jnp
from jax import lax
from jax.experimental import pallas as pl
from jax.experimental.pallas import tpu as pltpu
from jax.experimental.pallas import tpu_sc as plsc

_N = 10000
_DEG = 32
_F = 128
_OUT_F = 128
_E = _N * _DEG

_NW = 32
_CH = 4
_EPC = _CH * _DEG
_NCHUNKS = _N // _CH
_NITER = 80
_SUP = 1
_NSUP = _NITER // _SUP

_LANES = 16
_FCH = _F // _LANES


def _agg_body(idx_hbm, val_hbm, x_hbm, y_hbm,
              idx0, val0, rows0, out0, semi0, semv0, semg0, semo0,
              idx1, val1, rows1, out1, semi1, semv1, semg1, semo1):
    wid = lax.axis_index("s") * 2 + lax.axis_index("c")

    def chunk_of(g):
        return jnp.minimum(jnp.minimum(g, _NITER - 1) * _NW + wid,
                           _NCHUNKS - 1)

    def start_i(s, idx_v, semi):
        for q in range(_SUP):
            pltpu.async_copy(
                idx_hbm.at[pl.ds(chunk_of(s * _SUP + q) * _EPC, _EPC)],
                idx_v.at[q], semi)

    def wait_i(idx_v, semi):
        for q in range(_SUP):
            pltpu.make_async_copy(idx_hbm.at[pl.ds(0, _EPC)],
                                  idx_v.at[q], semi).wait()

    def start_v(s, val_v, semv):
        for q in range(_SUP):
            pltpu.async_copy(
                val_hbm.at[pl.ds(chunk_of(s * _SUP + q) * _EPC, _EPC)],
                val_v.at[q], semv)

    def wait_v(val_v, semv):
        for q in range(_SUP):
            pltpu.make_async_copy(val_hbm.at[pl.ds(0, _EPC)],
                                  val_v.at[q], semv).wait()

    def start_g(idx_v, rows_v, semg):
        for q in range(_SUP):
            pltpu.async_copy(x_hbm.at[idx_v.at[q]],
                             rows_v.at[pl.ds(q * _EPC, _EPC)], semg)

    def wait_g(idx_v, rows_v, semg):
        for q in range(_SUP):
            pltpu.make_async_copy(x_hbm.at[idx_v.at[q]],
                                  rows_v.at[pl.ds(q * _EPC, _EPC)],
                                  semg).wait()

    def start_o(s, out_v, semo):
        for q in range(_SUP):
            pltpu.async_copy(
                out_v.at[pl.ds(q * _CH, _CH)],
                y_hbm.at[pl.ds(chunk_of(s * _SUP + q) * _CH, _CH)], semo)

    def wait_o(out_v, semo):
        for q in range(_SUP):
            pltpu.make_async_copy(out_v.at[pl.ds(0, _CH)],
                                  y_hbm.at[pl.ds(0, _CH)], semo).wait()

    def bcast_gather(src, idx):
        return lax.gather(
            src, idx,
            dimension_numbers=lax.GatherDimensionNumbers(
                offset_dims=(), collapsed_slice_dims=(0,),
                start_index_map=(0,)),
            slice_sizes=(1,),
            mode=lax.GatherScatterMode.PROMISE_IN_BOUNDS)

    def compute(rows_v, val_v, out_v):
        for q in range(_SUP):
            for n in range(_CH):
                def group(h, accs, q=q, n=n):
                    vv = val_v[q, pl.ds(n * _DEG + (h // 2) * _LANES,
                                        _LANES)]
                    sub = (h % 2) * 8
                    base = n * _DEG + h * 8
                    for k in range(8):
                        v = bcast_gather(
                            vv, jnp.full((_LANES, 1), sub + k, jnp.int32))
                        j = q * _EPC + base + k
                        accs = tuple(
                            accs[fc]
                            + v * rows_v[j, pl.ds(fc * _LANES, _LANES)]
                            for fc in range(_FCH))
                    return accs

                accs = lax.fori_loop(
                    0, _DEG // 8, group,
                    tuple(jnp.zeros((_LANES,), jnp.float32)
                          for _ in range(_FCH)))
                for fc in range(_FCH):
                    out_v[q * _CH + n, pl.ds(fc * _LANES, _LANES)] = accs[fc]

    sets = ((idx0, val0, rows0, out0, semi0, semv0, semg0, semo0),
            (idx1, val1, rows1, out1, semi1, semv1, semg1, semo1))

    start_i(0, idx0, semi0)
    start_i(1, idx1, semi1)
    start_v(0, val0, semv0)
    start_v(1, val1, semv1)
    wait_i(idx0, semi0)
    start_g(idx0, rows0, semg0)

    def step(s, a, b):
        idx_a, val_a, rows_a, out_a, semi_a, semv_a, semg_a, semo_a = a
        idx_b, val_b, rows_b, out_b, semi_b, semv_b, semg_b, semo_b = b
        wait_i(idx_b, semi_b)
        start_g(idx_b, rows_b, semg_b)
        wait_g(idx_a, rows_a, semg_a)
        start_i(s + 2, idx_a, semi_a)
        wait_v(val_a, semv_a)

        @pl.when(s >= 2)
        def _():
            wait_o(out_a, semo_a)

        compute(rows_a, val_a, out_a)
        start_o(s, out_a, semo_a)
        start_v(s + 2, val_a, semv_a)

    @pl.loop(0, _NSUP, step=2)
    def _(s):
        step(s, sets[0], sets[1])
        step(s + 1, sets[1], sets[0])

    wait_g(idx0, rows0, semg0)
    wait_i(idx1, semi1)
    wait_v(val0, semv0)
    wait_v(val1, semv1)
    wait_o(out0, semo0)
    wait_o(out1, semo1)


@jax.jit
def _aggregate(col_idx, values, X):
    mesh = plsc.VectorSubcoreMesh(core_axis_name="c", subcore_axis_name="s")
    cp = pltpu.CompilerParams()
    if "needs_layout_passes" in pltpu.CompilerParams.__dataclass_fields__:
        cp = dataclasses.replace(cp, needs_layout_passes=False)
    buf_set = [
        pltpu.VMEM((_SUP, _EPC), jnp.int32),
        pltpu.VMEM((_SUP, _EPC), jnp.float32),
        pltpu.VMEM((_SUP * _EPC, _F), jnp.float32),
        pltpu.VMEM((_SUP * _CH, _F), jnp.float32),
        pltpu.SemaphoreType.DMA,
        pltpu.SemaphoreType.DMA,
        pltpu.SemaphoreType.DMA,
        pltpu.SemaphoreType.DMA,
    ]
    return pl.kernel(
        _agg_body,
        out_type=jax.ShapeDtypeStruct((_N, _F), jnp.float32),
        mesh=mesh,
        scratch_types=buf_set + buf_set,
        compiler_params=cp,
    )(col_idx, values, X)


def _mm_body(y_ref, w_ref, o_ref):
    o_ref[...] = jnp.dot(y_ref[...], w_ref[...],
                         preferred_element_type=jnp.float32,
                         precision=lax.Precision.HIGHEST)


_MB = 2000


@jax.jit
def _matmul(Y, W):
    return pl.pallas_call(
        _mm_body,
        grid=(_N // _MB,),
        in_specs=[
            pl.BlockSpec((_MB, _F), lambda i: (i, 0)),
            pl.BlockSpec((_F, _OUT_F), lambda i: (0, 0)),
        ],
        out_specs=pl.BlockSpec((_MB, _OUT_F), lambda i: (i, 0)),
        out_shape=jax.ShapeDtypeStruct((_N, _OUT_F), jnp.float32),
    )(Y, W)


def kernel(row_ptr, col_idx, values, X, num_neighbors, W):
    Y = _aggregate(col_idx, values, X)
    return _matmul(Y, W)
